# baseline (speedup 1.0000x reference)
import functools

import numpy as np
import jax
import jax.numpy as jnp
from jax import lax
from jax.experimental import pallas as pl
from jax.experimental.pallas import tpu as pltpu
from jax.experimental.pallas import tpu_sc as plsc

_N = 10000
_NP = 10240
_E = 320000
_H = 8
_DH = 8
_HID = 64
_C = 128
_NW = 32
_CHW = 79
_CH0 = 136
_EPAD = _NW * _CHW * _C
_NCH = _NW * _CHW
_BLK = 1024
_GRID = _NP // _BLK

_P_NP = np.zeros((_HID, _HID), np.float32)
for _h in range(_H):
    for _d in range(_DH):
        _P_NP[_h * _DH + _d, _d * _H + _h] = 1.0


def _att_cols(a):
    return (a[:, :, None] * jnp.eye(_H, dtype=jnp.float32)[:, None, :]).reshape(
        _HID, _H)


def _layer_mats(W_eff, b, a_src_sp, a_dst_sp, a_src_si, a_dst_si):
    P = jnp.asarray(_P_NP)
    cols = []
    for a_s, a_d in ((a_src_sp, a_dst_sp), (a_src_si, a_dst_si)):
        As, Ad = _att_cols(a_s), _att_cols(a_d)
        cols.append(jnp.concatenate([P, As, As], axis=1))
        cols.append(jnp.concatenate([Ad, Ad], axis=1))
    cols = jnp.concatenate([cols[0], cols[1], cols[2], cols[3]], axis=1)
    return W_eff @ cols, (b @ cols).reshape(1, 192)


def _pack_edges(ei):
    src = jnp.concatenate(
        [ei[0], jnp.zeros((_EPAD - _E,), jnp.int32)])
    dst = jnp.concatenate(
        [ei[1], jnp.full((_EPAD - _E,), _N, jnp.int32)])
    return jnp.stack([src.reshape(_NCH, _C), dst.reshape(_NCH, _C)], axis=1)


def _tables_body(x_ref, m_ref, b_ref, xa_sp, ad_sp, xa_si, ad_si):
    r = jnp.dot(x_ref[...], m_ref[...],
                preferred_element_type=jnp.float32) + b_ref[...]
    xa_sp[...] = r[:, 0:80]
    ad_sp[...] = r[:, 80:96]
    xa_si[...] = r[:, 96:176]
    ad_si[...] = r[:, 176:192]


def _combine_tables_body(osp_ref, osi_ref, at_ref, m_ref, b_ref,
                         xa_sp, ad_sp, xa_si, ad_si):
    a = at_ref[...]
    h = jnp.maximum(a[0:1, 0:1] * osp_ref[...] + a[0:1, 1:2] * osi_ref[...],
                    0.0)
    r = jnp.dot(h, m_ref[...], preferred_element_type=jnp.float32) + b_ref[...]
    xa_sp[...] = r[:, 0:80]
    ad_sp[...] = r[:, 80:96]
    xa_si[...] = r[:, 96:176]
    ad_si[...] = r[:, 176:192]


def _final_body(osp_ref, osi_ref, at_ref, m_ref, b_ref, out_ref):
    a = at_ref[...]
    h = jnp.maximum(a[0:1, 0:1] * osp_ref[...] + a[0:1, 1:2] * osi_ref[...],
                    0.0)
    out_ref[...] = jnp.dot(h, m_ref[...],
                           preferred_element_type=jnp.float32) + b_ref[...]


def _table_specs():
    return [
        pl.BlockSpec((_BLK, 80), lambda i: (i, 0)),
        pl.BlockSpec((_BLK, 16), lambda i: (i, 0)),
        pl.BlockSpec((_BLK, 80), lambda i: (i, 0)),
        pl.BlockSpec((_BLK, 16), lambda i: (i, 0)),
    ]


def _table_shapes():
    return [
        jax.ShapeDtypeStruct((_NP, 80), jnp.float32),
        jax.ShapeDtypeStruct((_NP, 16), jnp.float32),
        jax.ShapeDtypeStruct((_NP, 80), jnp.float32),
        jax.ShapeDtypeStruct((_NP, 16), jnp.float32),
    ]


def _tables_from_x(xp, M, brow):
    din = xp.shape[1]
    return pl.pallas_call(
        _tables_body,
        grid=(_GRID,),
        in_specs=[
            pl.BlockSpec((_BLK, din), lambda i: (i, 0)),
            pl.BlockSpec((din, 192), lambda i: (0, 0)),
            pl.BlockSpec((1, 192), lambda i: (0, 0)),
        ],
        out_specs=_table_specs(),
        out_shape=_table_shapes(),
    )(xp, M, brow)


def _tables_from_combine(o_sp, o_si, attn, M, brow):
    return pl.pallas_call(
        _combine_tables_body,
        grid=(_GRID,),
        in_specs=[
            pl.BlockSpec((_BLK, _HID), lambda i: (i, 0)),
            pl.BlockSpec((_BLK, _HID), lambda i: (i, 0)),
            pl.BlockSpec((1, 2), lambda i: (0, 0)),
            pl.BlockSpec((_HID, 192), lambda i: (0, 0)),
            pl.BlockSpec((1, 192), lambda i: (0, 0)),
        ],
        out_specs=_table_specs(),
        out_shape=_table_shapes(),
    )(o_sp, o_si, attn, M, brow)


def _final_linear(o_sp, o_si, attn, M, brow):
    nout = M.shape[1]
    return pl.pallas_call(
        _final_body,
        grid=(_GRID,),
        in_specs=[
            pl.BlockSpec((_BLK, _HID), lambda i: (i, 0)),
            pl.BlockSpec((_BLK, _HID), lambda i: (i, 0)),
            pl.BlockSpec((1, 2), lambda i: (0, 0)),
            pl.BlockSpec((_HID, nout), lambda i: (0, 0)),
            pl.BlockSpec((1, nout), lambda i: (0, 0)),
        ],
        out_specs=pl.BlockSpec((_BLK, nout), lambda i: (i, 0)),
        out_shape=jax.ShapeDtypeStruct((_NP, nout), jnp.float32),
    )(o_sp, o_si, attn, M, brow)


def _epilogue_body(acc_ref, kw_ref, kb_ref, q_ref,
                   osp_ref, osi_ref, attn_ref, ssum_ref):
    i = pl.program_id(0)
    nb = pl.num_programs(0)
    rid = lax.broadcasted_iota(jnp.int32, (_BLK, 1), 0) + i * _BLK
    mask = (rid < _N).astype(jnp.float32)
    for t in range(2):
        num = acc_ref[0, t, :, 0:64] + acc_ref[1, t, :, 0:64]
        den = acc_ref[0, t, :, 64:72] + acc_ref[1, t, :, 64:72]
        denb = jnp.tile(den, (1, _DH))
        o = jnp.maximum(num / (denb + 1e-16), 0.0)
        if t == 0:
            osp_ref[...] = o
        else:
            osi_ref[...] = o
        s = jnp.tanh(jnp.dot(o, kw_ref[...],
                             preferred_element_type=jnp.float32) + kb_ref[...])
        part = jnp.sum(s * mask, axis=0, keepdims=True)

        @pl.when(i == 0)
        def _():
            ssum_ref[pl.ds(t, 1), :] = part

        @pl.when(i > 0)
        def _():
            ssum_ref[pl.ds(t, 1), :] = ssum_ref[pl.ds(t, 1), :] + part

    @pl.when(i == nb - 1)
    def _():
        sv = jnp.sum(ssum_ref[...] * q_ref[...], axis=1) / _N
        e = jnp.exp(sv - jnp.max(sv))
        attn_ref[...] = (e / jnp.sum(e)).reshape(1, 2)


def _epilogue(acc, kWd, kb, q):
    return pl.pallas_call(
        _epilogue_body,
        grid=(_GRID,),
        in_specs=[
            pl.BlockSpec((2, 2, _BLK, 80), lambda i: (0, 0, i, 0)),
            pl.BlockSpec((_HID, _HID), lambda i: (0, 0)),
            pl.BlockSpec((1, _HID), lambda i: (0, 0)),
            pl.BlockSpec((1, _HID), lambda i: (0, 0)),
        ],
        out_specs=[
            pl.BlockSpec((_BLK, _HID), lambda i: (i, 0)),
            pl.BlockSpec((_BLK, _HID), lambda i: (i, 0)),
            pl.BlockSpec((1, 2), lambda i: (0, 0)),
        ],
        out_shape=[
            jax.ShapeDtypeStruct((_NP, _HID), jnp.float32),
            jax.ShapeDtypeStruct((_NP, _HID), jnp.float32),
            jax.ShapeDtypeStruct((1, 2), jnp.float32),
        ],
        scratch_shapes=[pltpu.VMEM((2, _HID), jnp.float32)],
    )(acc, kWd, kb, q)


def _sc_edge_body(xa_sp, ad_sp, xa_si, ad_si, eip_sp, eip_si, out,
                  acc_sp, acc_si, idx, rs, rd, sem_i, sem_g, sem_s):
    c = lax.axis_index("c")
    s = lax.axis_index("s")
    w = s * 2 + c
    rows_per_sub = _NP // 16

    @pl.loop(0, _C)
    def _(e):
        for k in range(5):
            rs[0, e, pl.ds(16 * k, 16)] = jnp.zeros((16,), jnp.float32)

    @pl.loop(0, rows_per_sub // _C)
    def _(j):
        base = s * rows_per_sub + j * _C
        pltpu.sync_copy(rs.at[0], acc_sp.at[pl.ds(base, _C)])
        pltpu.sync_copy(rs.at[0], acc_si.at[pl.ds(base, _C)])

    plsc.subcore_barrier()

    for t in range(2):
        xa = xa_sp if t == 0 else xa_si
        ad = ad_sp if t == 0 else ad_si
        eip = eip_sp if t == 0 else eip_si
        acc = acc_sp if t == 0 else acc_si
        n_c = jnp.where(c == 0, _CH0, 2 * _CHW - _CH0)
        base = s * (2 * _CHW) + jnp.where(c == 0, 0, _CH0)

        def start_idx(j):
            pltpu.async_copy(eip.at[base + j], idx.at[j % 3], sem_i)

        def wait_idx(j):
            pltpu.make_async_copy(eip.at[base + j], idx.at[j % 3],
                                  sem_i).wait()

        def start_g(j):
            b, i3 = j & 1, j % 3
            pltpu.async_copy(xa.at[idx.at[i3, 0]], rs.at[b], sem_g)
            pltpu.async_copy(ad.at[idx.at[i3, 1]], rd.at[b], sem_g)

        def wait_g(j):
            b, i3 = j & 1, j % 3
            pltpu.make_async_copy(xa.at[idx.at[i3, 0]], rs.at[b],
                                  sem_g).wait()
            pltpu.make_async_copy(ad.at[idx.at[i3, 1]], rd.at[b],
                                  sem_g).wait()

        def start_s(j):
            b = j & 1
            pltpu.async_copy(rs.at[b], acc.at[idx.at[j % 3, 1]], sem_s,
                             add=True)

        def wait_s(j):
            b = j & 1
            pltpu.make_async_copy(rs.at[b], acc.at[idx.at[j % 3, 1]],
                                  sem_s).wait()

        start_idx(0)
        wait_idx(0)
        start_g(0)
        start_idx(1)

        @pl.loop(0, n_c)
        def _(j):
            b = j & 1
            wait_g(j)

            @pl.when(j >= 1)
            def _():
                wait_s(j - 1)

            @pl.when(j + 1 < n_c)
            def _():
                wait_idx(j + 1)
                start_g(j + 1)

            @pl.when(j + 2 < n_c)
            def _():
                start_idx(j + 2)

            @plsc.parallel_loop(0, _C, 1, unroll=4)
            def _(e):
                z = rs[b, e, pl.ds(64, 16)] + rd[b, e, :]
                ae = jnp.exp(jnp.maximum(z, z * 0.2))
                rs[b, e, pl.ds(64, 16)] = ae
                for k in range(4):
                    rs[b, e, pl.ds(16 * k, 16)] = (
                        rs[b, e, pl.ds(16 * k, 16)] * ae)

            start_s(j)

        wait_s(n_c - 1)

    plsc.subcore_barrier()

    base = s * rows_per_sub
    pltpu.sync_copy(acc_sp.at[pl.ds(base, rows_per_sub)],
                    out.at[c, 0, pl.ds(base, rows_per_sub)])
    pltpu.sync_copy(acc_si.at[pl.ds(base, rows_per_sub)],
                    out.at[c, 1, pl.ds(base, rows_per_sub)])


@functools.lru_cache(maxsize=None)
def _sc_kernel():
    return pl.kernel(
        _sc_edge_body,
        mesh=plsc.VectorSubcoreMesh(core_axis_name="c", subcore_axis_name="s"),
        compiler_params=pltpu.CompilerParams(use_tc_tiling_on_sc=False),
        out_type=jax.ShapeDtypeStruct((2, 2, _NP, 80), jnp.float32),
        scratch_types=[
            pltpu.VMEM_SHARED((_NP, 80), jnp.float32),
            pltpu.VMEM_SHARED((_NP, 80), jnp.float32),
            pltpu.VMEM((3, 2, _C), jnp.int32),
            pltpu.VMEM((2, _C, 80), jnp.float32),
            pltpu.VMEM((2, _C, 16), jnp.float32),
            pltpu.SemaphoreType.DMA,
            pltpu.SemaphoreType.DMA,
            pltpu.SemaphoreType.DMA,
        ],
    )


def _sc_edge_pass(*args):
    return _sc_kernel()(*args)


def _layer(tables, eip_sp, eip_si, kWd, kb, q):
    xa_sp, ad_sp, xa_si, ad_si = tables
    acc = _sc_edge_pass(xa_sp, ad_sp, xa_si, ad_si, eip_sp, eip_si)
    return _epilogue(acc, kWd, kb.reshape(1, _HID), q.reshape(1, _HID))


def kernel(x, ei_spatial, ei_similar, proj_W1, proj_b1, a_src_sp1, a_dst_sp1,
           a_src_si1, a_dst_si1, k_W1, k_b1, q1, proj_W2, proj_b2, a_src_sp2,
           a_dst_sp2, a_src_si2, a_dst_si2, k_W2, k_b2, q2, lin_W, lin_b):
    P = jnp.asarray(_P_NP)
    M1, b1row = _layer_mats(proj_W1, proj_b1,
                            a_src_sp1, a_dst_sp1, a_src_si1, a_dst_si1)
    M2, b2row = _layer_mats(P.T @ proj_W2, proj_b2,
                            a_src_sp2, a_dst_sp2, a_src_si2, a_dst_si2)
    kWd1, kWd2 = P.T @ k_W1, P.T @ k_W2
    linWd = P.T @ lin_W

    eip_sp = _pack_edges(ei_spatial)
    eip_si = _pack_edges(ei_similar)
    xp = jnp.pad(x, ((0, _NP - _N), (0, 0)))

    tables1 = _tables_from_x(xp, M1, b1row)
    o_sp1, o_si1, attn1 = _layer(tables1, eip_sp, eip_si, kWd1, k_b1, q1)

    tables2 = _tables_from_combine(o_sp1, o_si1, attn1, M2, b2row)
    o_sp2, o_si2, attn2 = _layer(tables2, eip_sp, eip_si, kWd2, k_b2, q2)

    out = _final_linear(o_sp2, o_si2, attn2, linWd, lin_b.reshape(1, -1))
    return out[:_N]


# speedup vs baseline: 1.0956x; 1.0956x over previous
import functools

import numpy as np
import jax
import jax.numpy as jnp
from jax import lax
from jax.experimental import pallas as pl
from jax.experimental.pallas import tpu as pltpu
from jax.experimental.pallas import tpu_sc as plsc

_N = 10000
_NP = 10240
_E = 320000
_H = 8
_DH = 8
_HID = 64
_C = 128
_NW = 32
_CHW = 79
_CH0 = 126
_EPAD = _NW * _CHW * _C
_NCH = _NW * _CHW
_BLK = 1024
_GRID = _NP // _BLK

_P_NP = np.zeros((_HID, _HID), np.float32)
for _h in range(_H):
    for _d in range(_DH):
        _P_NP[_h * _DH + _d, _d * _H + _h] = 1.0


def _att_cols(a):
    return (a[:, :, None] * jnp.eye(_H, dtype=jnp.float32)[:, None, :]).reshape(
        _HID, _H)


def _layer_mats(W_eff, b, a_src_sp, a_dst_sp, a_src_si, a_dst_si):
    P = jnp.asarray(_P_NP)
    cols = []
    for a_s, a_d in ((a_src_sp, a_dst_sp), (a_src_si, a_dst_si)):
        As, Ad = _att_cols(a_s), _att_cols(a_d)
        cols.append(jnp.concatenate([P, As, As], axis=1))
        cols.append(jnp.concatenate([Ad, Ad], axis=1))
    cols = jnp.concatenate([cols[0], cols[1], cols[2], cols[3]], axis=1)
    return W_eff @ cols, (b @ cols).reshape(1, 192)


def _pack_edges(ei):
    src = jnp.concatenate(
        [ei[0], jnp.zeros((_EPAD - _E,), jnp.int32)])
    dst = jnp.concatenate(
        [ei[1], jnp.full((_EPAD - _E,), _N, jnp.int32)])
    return jnp.stack([src.reshape(_NCH, _C), dst.reshape(_NCH, _C)], axis=1)


def _tables_body(x_ref, m_ref, b_ref, xa_sp, ad_sp, xa_si, ad_si):
    r = jnp.dot(x_ref[...], m_ref[...],
                preferred_element_type=jnp.float32) + b_ref[...]
    xa_sp[...] = r[:, 0:80]
    ad_sp[...] = r[:, 80:96]
    xa_si[...] = r[:, 96:176]
    ad_si[...] = r[:, 176:192]


def _combine_tables_body(osp_ref, osi_ref, at_ref, m_ref, b_ref,
                         xa_sp, ad_sp, xa_si, ad_si):
    a = at_ref[...]
    h = jnp.maximum(a[0:1, 0:1] * osp_ref[...] + a[0:1, 1:2] * osi_ref[...],
                    0.0)
    r = jnp.dot(h, m_ref[...], preferred_element_type=jnp.float32) + b_ref[...]
    xa_sp[...] = r[:, 0:80]
    ad_sp[...] = r[:, 80:96]
    xa_si[...] = r[:, 96:176]
    ad_si[...] = r[:, 176:192]


def _final_body(osp_ref, osi_ref, at_ref, m_ref, b_ref, out_ref):
    a = at_ref[...]
    h = jnp.maximum(a[0:1, 0:1] * osp_ref[...] + a[0:1, 1:2] * osi_ref[...],
                    0.0)
    out_ref[...] = jnp.dot(h, m_ref[...],
                           preferred_element_type=jnp.float32) + b_ref[...]


def _table_specs():
    return [
        pl.BlockSpec((_BLK, 80), lambda i: (i, 0)),
        pl.BlockSpec((_BLK, 16), lambda i: (i, 0)),
        pl.BlockSpec((_BLK, 80), lambda i: (i, 0)),
        pl.BlockSpec((_BLK, 16), lambda i: (i, 0)),
    ]


def _table_shapes():
    return [
        jax.ShapeDtypeStruct((_NP, 80), jnp.float32),
        jax.ShapeDtypeStruct((_NP, 16), jnp.float32),
        jax.ShapeDtypeStruct((_NP, 80), jnp.float32),
        jax.ShapeDtypeStruct((_NP, 16), jnp.float32),
    ]


def _tables_from_x(xp, M, brow):
    din = xp.shape[1]
    return pl.pallas_call(
        _tables_body,
        grid=(_GRID,),
        in_specs=[
            pl.BlockSpec((_BLK, din), lambda i: (i, 0)),
            pl.BlockSpec((din, 192), lambda i: (0, 0)),
            pl.BlockSpec((1, 192), lambda i: (0, 0)),
        ],
        out_specs=_table_specs(),
        out_shape=_table_shapes(),
    )(xp, M, brow)


def _tables_from_combine(o_sp, o_si, attn, M, brow):
    return pl.pallas_call(
        _combine_tables_body,
        grid=(_GRID,),
        in_specs=[
            pl.BlockSpec((_BLK, _HID), lambda i: (i, 0)),
            pl.BlockSpec((_BLK, _HID), lambda i: (i, 0)),
            pl.BlockSpec((1, 2), lambda i: (0, 0)),
            pl.BlockSpec((_HID, 192), lambda i: (0, 0)),
            pl.BlockSpec((1, 192), lambda i: (0, 0)),
        ],
        out_specs=_table_specs(),
        out_shape=_table_shapes(),
    )(o_sp, o_si, attn, M, brow)


def _final_linear(o_sp, o_si, attn, M, brow):
    nout = M.shape[1]
    return pl.pallas_call(
        _final_body,
        grid=(_GRID,),
        in_specs=[
            pl.BlockSpec((_BLK, _HID), lambda i: (i, 0)),
            pl.BlockSpec((_BLK, _HID), lambda i: (i, 0)),
            pl.BlockSpec((1, 2), lambda i: (0, 0)),
            pl.BlockSpec((_HID, nout), lambda i: (0, 0)),
            pl.BlockSpec((1, nout), lambda i: (0, 0)),
        ],
        out_specs=pl.BlockSpec((_BLK, nout), lambda i: (i, 0)),
        out_shape=jax.ShapeDtypeStruct((_NP, nout), jnp.float32),
    )(o_sp, o_si, attn, M, brow)


def _epilogue_body(acc_ref, kw_ref, kb_ref, q_ref,
                   osp_ref, osi_ref, attn_ref, ssum_ref):
    i = pl.program_id(0)
    nb = pl.num_programs(0)
    rid = lax.broadcasted_iota(jnp.int32, (_BLK, 1), 0) + i * _BLK
    mask = (rid < _N).astype(jnp.float32)
    for t in range(2):
        num = acc_ref[0, t, :, 0:64] + acc_ref[1, t, :, 0:64]
        den = acc_ref[0, t, :, 64:72] + acc_ref[1, t, :, 64:72]
        denb = jnp.tile(den, (1, _DH))
        o = jnp.maximum(num / (denb + 1e-16), 0.0)
        if t == 0:
            osp_ref[...] = o
        else:
            osi_ref[...] = o
        s = jnp.tanh(jnp.dot(o, kw_ref[...],
                             preferred_element_type=jnp.float32) + kb_ref[...])
        part = jnp.sum(s * mask, axis=0, keepdims=True)

        @pl.when(i == 0)
        def _():
            ssum_ref[pl.ds(t, 1), :] = part

        @pl.when(i > 0)
        def _():
            ssum_ref[pl.ds(t, 1), :] = ssum_ref[pl.ds(t, 1), :] + part

    @pl.when(i == nb - 1)
    def _():
        sv = jnp.sum(ssum_ref[...] * q_ref[...], axis=1) / _N
        e = jnp.exp(sv - jnp.max(sv))
        attn_ref[...] = (e / jnp.sum(e)).reshape(1, 2)


def _epilogue(acc, kWd, kb, q):
    return pl.pallas_call(
        _epilogue_body,
        grid=(_GRID,),
        in_specs=[
            pl.BlockSpec((2, 2, _BLK, 80), lambda i: (0, 0, i, 0)),
            pl.BlockSpec((_HID, _HID), lambda i: (0, 0)),
            pl.BlockSpec((1, _HID), lambda i: (0, 0)),
            pl.BlockSpec((1, _HID), lambda i: (0, 0)),
        ],
        out_specs=[
            pl.BlockSpec((_BLK, _HID), lambda i: (i, 0)),
            pl.BlockSpec((_BLK, _HID), lambda i: (i, 0)),
            pl.BlockSpec((1, 2), lambda i: (0, 0)),
        ],
        out_shape=[
            jax.ShapeDtypeStruct((_NP, _HID), jnp.float32),
            jax.ShapeDtypeStruct((_NP, _HID), jnp.float32),
            jax.ShapeDtypeStruct((1, 2), jnp.float32),
        ],
        scratch_shapes=[pltpu.VMEM((2, _HID), jnp.float32)],
    )(acc, kWd, kb, q)


def _sc_edge_body(xa_sp, ad_sp, xa_si, ad_si, eip_sp, eip_si, out,
                  acc_sp, acc_si, idx, rs, rd, sem_i, sem_g, sem_s):
    c = lax.axis_index("c")
    s = lax.axis_index("s")
    w = s * 2 + c
    rows_per_sub = _NP // 16

    @pl.loop(0, _C)
    def _(e):
        for k in range(5):
            rs[0, e, pl.ds(16 * k, 16)] = jnp.zeros((16,), jnp.float32)

    @pl.loop(0, rows_per_sub // _C)
    def _(j):
        base = s * rows_per_sub + j * _C
        pltpu.sync_copy(rs.at[0], acc_sp.at[pl.ds(base, _C)])
        pltpu.sync_copy(rs.at[0], acc_si.at[pl.ds(base, _C)])

    plsc.subcore_barrier()

    for t in range(2):
        xa = xa_sp if t == 0 else xa_si
        ad = ad_sp if t == 0 else ad_si
        eip = eip_sp if t == 0 else eip_si
        acc = acc_sp if t == 0 else acc_si
        n_c = jnp.where(c == 0, _CH0, 2 * _CHW - _CH0)
        base = s * (2 * _CHW) + jnp.where(c == 0, 0, _CH0)

        def start_idx(j):
            pltpu.async_copy(eip.at[base + j], idx.at[j % 3], sem_i)

        def wait_idx(j):
            pltpu.make_async_copy(eip.at[base + j], idx.at[j % 3],
                                  sem_i).wait()

        def start_g(j):
            b, i3 = j & 1, j % 3
            pltpu.async_copy(xa.at[idx.at[i3, 0]], rs.at[b], sem_g)
            pltpu.async_copy(ad.at[idx.at[i3, 1]], rd.at[b], sem_g)

        def wait_g(j):
            b, i3 = j & 1, j % 3
            pltpu.make_async_copy(xa.at[idx.at[i3, 0]], rs.at[b],
                                  sem_g).wait()
            pltpu.make_async_copy(ad.at[idx.at[i3, 1]], rd.at[b],
                                  sem_g).wait()

        def start_s(j):
            b = j & 1
            pltpu.async_copy(rs.at[b], acc.at[idx.at[j % 3, 1]], sem_s,
                             add=True)

        def wait_s(j):
            b = j & 1
            pltpu.make_async_copy(rs.at[b], acc.at[idx.at[j % 3, 1]],
                                  sem_s).wait()

        start_idx(0)
        wait_idx(0)
        start_g(0)
        start_idx(1)

        @pl.loop(0, n_c)
        def _(j):
            b = j & 1
            wait_g(j)

            @pl.when(j >= 1)
            def _():
                wait_s(j - 1)

            @pl.when(j + 1 < n_c)
            def _():
                wait_idx(j + 1)
                start_g(j + 1)

            @pl.when(j + 2 < n_c)
            def _():
                start_idx(j + 2)

            @plsc.parallel_loop(0, _C, 1, unroll=4)
            def _(e):
                z = rs[b, e, pl.ds(64, 16)] + rd[b, e, :]
                ae = jnp.exp(jnp.maximum(z, z * 0.2))
                rs[b, e, pl.ds(64, 16)] = ae
                for k in range(4):
                    rs[b, e, pl.ds(16 * k, 16)] = (
                        rs[b, e, pl.ds(16 * k, 16)] * ae)

            start_s(j)

        wait_s(n_c - 1)

    plsc.subcore_barrier()

    base = s * rows_per_sub
    pltpu.sync_copy(acc_sp.at[pl.ds(base, rows_per_sub)],
                    out.at[c, 0, pl.ds(base, rows_per_sub)])
    pltpu.sync_copy(acc_si.at[pl.ds(base, rows_per_sub)],
                    out.at[c, 1, pl.ds(base, rows_per_sub)])


@functools.lru_cache(maxsize=None)
def _sc_kernel():
    return pl.kernel(
        _sc_edge_body,
        mesh=plsc.VectorSubcoreMesh(core_axis_name="c", subcore_axis_name="s"),
        compiler_params=pltpu.CompilerParams(use_tc_tiling_on_sc=False),
        out_type=jax.ShapeDtypeStruct((2, 2, _NP, 80), jnp.float32),
        scratch_types=[
            pltpu.VMEM_SHARED((_NP, 80), jnp.float32),
            pltpu.VMEM_SHARED((_NP, 80), jnp.float32),
            pltpu.VMEM((3, 2, _C), jnp.int32),
            pltpu.VMEM((2, _C, 80), jnp.float32),
            pltpu.VMEM((2, _C, 16), jnp.float32),
            pltpu.SemaphoreType.DMA,
            pltpu.SemaphoreType.DMA,
            pltpu.SemaphoreType.DMA,
        ],
    )


def _sc_edge_pass(*args):
    return _sc_kernel()(*args)


def _layer(tables, eip_sp, eip_si, kWd, kb, q):
    xa_sp, ad_sp, xa_si, ad_si = tables
    acc = _sc_edge_pass(xa_sp, ad_sp, xa_si, ad_si, eip_sp, eip_si)
    return _epilogue(acc, kWd, kb.reshape(1, _HID), q.reshape(1, _HID))


def kernel(x, ei_spatial, ei_similar, proj_W1, proj_b1, a_src_sp1, a_dst_sp1,
           a_src_si1, a_dst_si1, k_W1, k_b1, q1, proj_W2, proj_b2, a_src_sp2,
           a_dst_sp2, a_src_si2, a_dst_si2, k_W2, k_b2, q2, lin_W, lin_b):
    P = jnp.asarray(_P_NP)
    M1, b1row = _layer_mats(proj_W1, proj_b1,
                            a_src_sp1, a_dst_sp1, a_src_si1, a_dst_si1)
    M2, b2row = _layer_mats(P.T @ proj_W2, proj_b2,
                            a_src_sp2, a_dst_sp2, a_src_si2, a_dst_si2)
    kWd1, kWd2 = P.T @ k_W1, P.T @ k_W2
    linWd = P.T @ lin_W

    eip_sp = _pack_edges(ei_spatial)
    eip_si = _pack_edges(ei_similar)
    xp = jnp.pad(x, ((0, _NP - _N), (0, 0)))

    tables1 = _tables_from_x(xp, M1, b1row)
    o_sp1, o_si1, attn1 = _layer(tables1, eip_sp, eip_si, kWd1, k_b1, q1)

    tables2 = _tables_from_combine(o_sp1, o_si1, attn1, M2, b2row)
    o_sp2, o_si2, attn2 = _layer(tables2, eip_sp, eip_si, kWd2, k_b2, q2)

    out = _final_linear(o_sp2, o_si2, attn2, linWd, lin_b.reshape(1, -1))
    return out[:_N]
